# baseline (speedup 1.0000x reference)
import jax
import jax.numpy as jnp
from jax.experimental import pallas as pl

F32 = jnp.float32


def _audio_kernel(x_ref, w1a_ref, w1b_ref, w1c_ref, b1_ref,
                  w2a_ref, w2b_ref, b2_ref,
                  w3a_ref, w3b_ref, w3c_ref, b3_ref,
                  feat_ref, ea_ref):
    x = x_ref[0]

    z8 = jnp.zeros((8, 512), F32)
    sd = jnp.concatenate([z8, x[:-8, :]], axis=0)
    su = jnp.concatenate([x[8:, :], z8], axis=0)
    z = (jnp.dot(sd, w1a_ref[...], preferred_element_type=F32)
         + jnp.dot(x, w1b_ref[...], preferred_element_type=F32)
         + jnp.dot(su, w1c_ref[...], preferred_element_type=F32)
         + b1_ref[...])

    v = z.reshape(100, 2, 4, 512)
    x2 = jax.nn.relu(jnp.maximum(v[:, 0], v[:, 1])).reshape(400, 512)

    v2 = x2.reshape(200, 2, 512)
    za = v2[:, 0, :]
    zb = v2[:, 1, :]
    x3 = jax.nn.relu(jnp.dot(za, w2a_ref[...], preferred_element_type=F32)
                     + jnp.dot(zb, w2b_ref[...], preferred_element_type=F32)
                     + b2_ref[...])

    z2r = jnp.zeros((2, 512), F32)
    sd3 = jnp.concatenate([z2r, x3[:-2, :]], axis=0)
    su3 = jnp.concatenate([x3[2:, :], z2r], axis=0)
    z3 = (jnp.dot(sd3, w3a_ref[...], preferred_element_type=F32)
          + jnp.dot(x3, w3b_ref[...], preferred_element_type=F32)
          + jnp.dot(su3, w3c_ref[...], preferred_element_type=F32)
          + b3_ref[...])

    v3 = z3.reshape(50, 2, 2, 512)
    feat = jax.nn.relu(jnp.maximum(v3[:, 0], v3[:, 1])).reshape(100, 512)
    feat_ref[0] = feat

    ea_ref[0] = jnp.max(feat.reshape(10, 10, 512), axis=1)


def _visual_kernel(x_ref, wv_ref, bv_ref, fv_ref, ev_ref):
    x = x_ref[...]
    n = x.shape[0]
    p = jax.lax.broadcasted_iota(jnp.int32, (n, 1), 0)
    s = p % 49
    h = s // 7
    w = s % 7

    acc = jnp.broadcast_to(bv_ref[...], (n, 512))
    t = 0
    for dh in (-1, 0, 1):
        for dw in (-1, 0, 1):
            sh = 7 * dh + dw
            if sh > 0:
                xs = jnp.concatenate([x[sh:, :], jnp.zeros((sh, 512), F32)], axis=0)
            elif sh < 0:
                xs = jnp.concatenate([jnp.zeros((-sh, 512), F32), x[:sh, :]], axis=0)
            else:
                xs = x
            contrib = jnp.dot(xs, wv_ref[t], preferred_element_type=F32)
            valid = ((h + dh >= 0) & (h + dh < 7) & (w + dw >= 0) & (w + dw < 7))
            acc = acc + jnp.where(valid, contrib, 0.0)
            t += 1

    fv = jax.nn.relu(acc)
    fv_ref[...] = fv
    ev_ref[...] = jnp.max(fv.reshape(32, 49, 512), axis=1)


def _mlp_kernel(ea_ref, ev_ref, wa_ref, wv_ref, b1_ref, w2_ref, b2_ref, out_ref):
    hidden = jax.nn.relu(jnp.dot(ea_ref[...], wa_ref[...], preferred_element_type=F32)
                         + jnp.dot(ev_ref[...], wv_ref[...], preferred_element_type=F32)
                         + b1_ref[...])
    out_ref[...] = jnp.dot(hidden, w2_ref[...], preferred_element_type=F32) + b2_ref[...]


def kernel(audio, visual, W1, g1, b1, W2, g2, b2, W3, g3, b3, Wv, gv, bv, D1w, D1b, D2w, D2b):
    s = (1.0 / jnp.sqrt(jnp.float32(1.0 + 1e-5)))
    s1 = g1 * s
    s2 = g2 * s
    s3 = g3 * s
    sv = gv * s

    w1 = W1[:, :, :, 0] * s1[:, None, None]
    w1a, w1b, w1c = (w1[:, :, t].T for t in range(3))
    w2 = W2[:, :, 0, :] * s2[:, None, None]
    w2a, w2b = (w2[:, :, t].T for t in range(2))
    w3 = W3[:, :, :, 0] * s3[:, None, None]
    w3a, w3b, w3c = (w3[:, :, t].T for t in range(3))
    wv = (Wv * sv[:, None, None, None]).reshape(512, 512, 9)
    wvt = jnp.transpose(wv, (2, 1, 0))

    b1r = b1.reshape(1, 512)
    b2r = b2.reshape(1, 512)
    b3r = b3.reshape(1, 512)
    bvr = bv.reshape(1, 512)

    at = jnp.transpose(audio, (0, 2, 3, 1)).reshape(16, 800, 512)
    wspec = pl.BlockSpec((512, 512), lambda i: (0, 0))
    bspec = pl.BlockSpec((1, 512), lambda i: (0, 0))
    feat_r, ea = pl.pallas_call(
        _audio_kernel,
        grid=(16,),
        in_specs=[pl.BlockSpec((1, 800, 512), lambda i: (i, 0, 0)),
                  wspec, wspec, wspec, bspec,
                  wspec, wspec, bspec,
                  wspec, wspec, wspec, bspec],
        out_specs=[pl.BlockSpec((1, 100, 512), lambda i: (i, 0, 0)),
                   pl.BlockSpec((1, 10, 512), lambda i: (i, 0, 0))],
        out_shape=[jax.ShapeDtypeStruct((16, 100, 512), F32),
                   jax.ShapeDtypeStruct((16, 10, 512), F32)],
    )(at, w1a, w1b, w1c, b1r, w2a, w2b, b2r, w3a, w3b, w3c, b3r)
    feat_a = jnp.transpose(feat_r.reshape(16, 50, 2, 512), (0, 3, 1, 2))

    vt = jnp.transpose(visual, (0, 2, 3, 1)).reshape(7840, 512)
    fv, ev = pl.pallas_call(
        _visual_kernel,
        grid=(5,),
        in_specs=[pl.BlockSpec((1568, 512), lambda i: (i, 0)),
                  pl.BlockSpec((9, 512, 512), lambda i: (0, 0, 0)),
                  pl.BlockSpec((1, 512), lambda i: (0, 0))],
        out_specs=[pl.BlockSpec((1568, 512), lambda i: (i, 0)),
                   pl.BlockSpec((32, 512), lambda i: (i, 0))],
        out_shape=[jax.ShapeDtypeStruct((7840, 512), F32),
                   jax.ShapeDtypeStruct((160, 512), F32)],
    )(vt, wvt, bvr)
    feat_v = jnp.transpose(fv.reshape(160, 49, 512), (0, 2, 1)).reshape(160, 512, 7, 7)

    ea2 = ea.reshape(160, 512)
    waT = D1w[:, :512].T
    wvT = D1w[:, 512:].T
    common = pl.pallas_call(
        _mlp_kernel,
        out_shape=jax.ShapeDtypeStruct((160, 2), F32),
    )(ea2, ev, waT, wvT, D1b.reshape(1, 128), D2w.T, D2b.reshape(1, 2))

    return (common.reshape(16, 10, 2), feat_a, feat_v)
